# baseline (speedup 1.0000x reference)
import functools

import jax
import jax.numpy as jnp
from jax import lax
from jax.experimental import pallas as pl
from jax.experimental.pallas import tpu as pltpu
from jax.experimental.pallas import tpu_sc as plsc

F = 26
B = 16384
D = 32
TOTAL = B * F
NC = 2
NS = 16
NW = NC * NS
PER_W = TOTAL // NW
IDXROW = 128
NJ = 13
CHUNK = NJ * IDXROW
NCH = PER_W // CHUNK
NV = IDXROW // 16
NG = NW * NCH


def _emb_body(x_hbm, off_hbm, table_hbm, out_hbm, obuf, xbuf, rows, sem):
    wid = lax.axis_index("s") * NC + lax.axis_index("c")
    pltpu.sync_copy(off_hbm, obuf)

    def chunk_body(c):
        g = wid * NCH + c
        pltpu.sync_copy(x_hbm.at[g], xbuf)
        for j in range(NJ):
            for v in range(NV):
                sl = pl.ds(v * 16, 16)
                xbuf[j, sl] = xbuf[j, sl] + obuf[j, sl]
        cps = [
            pltpu.async_copy(
                table_hbm.at[xbuf.at[j]],
                rows.at[pl.ds(j * IDXROW, IDXROW)],
                sem,
            )
            for j in range(NJ)
        ]
        for cp in cps:
            cp.wait()
        pltpu.sync_copy(rows, out_hbm.at[g])

    pl.loop(0, NCH)(chunk_body)


@jax.jit
def _emb_call(x3, off2, table):
    mesh = plsc.VectorSubcoreMesh(
        core_axis_name="c", subcore_axis_name="s", num_cores=NC, num_subcores=NS
    )
    return pl.kernel(
        _emb_body,
        out_type=jax.ShapeDtypeStruct((NG, CHUNK, D), jnp.float32),
        mesh=mesh,
        scratch_types=[
            pltpu.VMEM((NJ, IDXROW), jnp.int32),
            pltpu.VMEM((NJ, IDXROW), jnp.int32),
            pltpu.VMEM((CHUNK, D), jnp.float32),
            pltpu.SemaphoreType.DMA,
        ],
        compiler_params=pltpu.CompilerParams(use_tc_tiling_on_sc=False),
    )(x3, off2, table)


def kernel(x, table, offsets):
    x3 = x.reshape(NG, NJ, IDXROW)
    off2 = jnp.tile(offsets, CHUNK // F).reshape(NJ, IDXROW)
    out = _emb_call(x3, off2, table)
    return out.reshape(B, F * D)


# speedup vs baseline: 1.0025x; 1.0025x over previous
import functools

import jax
import jax.numpy as jnp
from jax import lax
from jax.experimental import pallas as pl
from jax.experimental.pallas import tpu as pltpu
from jax.experimental.pallas import tpu_sc as plsc

F = 26
B = 16384
D = 32
TOTAL = B * F
NC = 2
NS = 16
NW = NC * NS
PER_W = TOTAL // NW
CHUNK = 3328
NCH = PER_W // CHUNK
NV = CHUNK // 16
NG = NW * NCH


def _emb_body(x_hbm, off_hbm, table_hbm, out_hbm, obuf, xbuf, rows, sem):
    wid = lax.axis_index("s") * NC + lax.axis_index("c")
    pltpu.sync_copy(off_hbm, obuf)

    def chunk_body(c):
        g = wid * NCH + c
        pltpu.sync_copy(x_hbm.at[g], xbuf)
        for v in range(NV):
            sl = pl.ds(v * 16, 16)
            xbuf[sl] = xbuf[sl] + obuf[sl]
        pltpu.async_copy(table_hbm.at[xbuf], rows, sem).wait()
        pltpu.sync_copy(rows, out_hbm.at[g])

    pl.loop(0, NCH)(chunk_body)


@jax.jit
def _emb_call(x2, off1, table):
    mesh = plsc.VectorSubcoreMesh(
        core_axis_name="c", subcore_axis_name="s", num_cores=NC, num_subcores=NS
    )
    return pl.kernel(
        _emb_body,
        out_type=jax.ShapeDtypeStruct((NG, CHUNK, D), jnp.float32),
        mesh=mesh,
        scratch_types=[
            pltpu.VMEM((CHUNK,), jnp.int32),
            pltpu.VMEM((CHUNK,), jnp.int32),
            pltpu.VMEM((CHUNK, D), jnp.float32),
            pltpu.SemaphoreType.DMA,
        ],
        compiler_params=pltpu.CompilerParams(use_tc_tiling_on_sc=False),
    )(x2, off1, table)


def kernel(x, table, offsets):
    x2 = x.reshape(NG, CHUNK)
    off1 = jnp.tile(offsets, CHUNK // F)
    out = _emb_call(x2, off1, table)
    return out.reshape(B, F * D)
